# baseline (speedup 1.0000x reference)
import functools

import jax
import jax.numpy as jnp
from jax import lax
from jax.experimental import pallas as pl
from jax.experimental.pallas import tpu as pltpu
from jax.experimental.pallas import tpu_sc as plsc

NUM_CLASSES = 1000000
EMBED_DIM = 64
BATCH = 16384

NC = 2
NS = 16
NW = NC * NS
NTB = (NUM_CLASSES + 127) // 128
TPW = (NTB + NW - 1) // NW
STAGE_ROWS = 20480


def _sc_body(tableT, idx_hbm, stage_hbm, all_idx, my_lab, my_pos, lab2, pos2,
             slab_v, rows_v, posw_v, starts_v, sem_ga, sem_gb, sem_oa, sem_ob):
    wid = lax.axis_index("s") * NC + lax.axis_index("c")
    lane = lax.iota(jnp.int32, 16)
    lo = wid * TPW
    hi = jnp.minimum(lo + TPW, NTB)
    pltpu.sync_copy(idx_hbm, all_idx)

    def sc_body(q, ptr):
        for k in range(4):
            off = q * 64 + k * 16
            vec = all_idx[pl.ds(off, 16)]
            tb = jnp.right_shift(vec, 7)
            m = jnp.logical_and(tb >= lo, tb < hi)
            plsc.store_compressed(my_lab.at[pl.ds(ptr, 16)], vec, mask=m)
            plsc.store_compressed(my_pos.at[pl.ds(ptr, 16)], lane + off,
                                  mask=m)
            ptr = ptr + plsc.all_reduce_population_count(m)[0]
        return ptr

    mcnt = lax.fori_loop(0, BATCH // 64, sc_body, jnp.int32(0))
    nwin = lax.div(mcnt + 15, jnp.int32(16))

    starts = []
    gptr = jnp.int32(0)
    for k in range(16):
        starts.append(gptr)

        def b1(w, p, k=k):
            woff = w * 16
            labv = my_lab[pl.ds(woff, 16)]
            posv = my_pos[pl.ds(woff, 16)]
            valid = (lane + woff) < mcnt
            bk = jnp.right_shift(jnp.right_shift(labv, 7) - lo, 4)
            m = jnp.logical_and(bk == k, valid)
            plsc.store_compressed(lab2.at[pl.ds(p, 16)], labv, mask=m)
            plsc.store_compressed(pos2.at[pl.ds(p, 16)], posv, mask=m)
            return p + plsc.all_reduce_population_count(m)[0]

        gptr = lax.fori_loop(0, nwin, b1, gptr)
    starts.append(gptr)

    sv = jnp.zeros((16,), jnp.int32)
    for k in range(16):
        sv = jnp.where(lane == k, starts[k], sv)
    starts_v[pl.ds(0, 16)] = sv
    starts_v[pl.ds(16, 16)] = jnp.where(lane == 0, starts[16], 0)

    def fire(t, b):
        off = pl.multiple_of(t * 128, 128)
        for par in range(2):
            @pl.when(b == par)
            def _(par=par):
                pltpu.async_copy(
                    tableT.at[:, pl.ds(off, 128)], slab_v.at[par],
                    [sem_ga, sem_gb][par])

    def drain_slab(b):
        for par in range(2):
            @pl.when(b == par)
            def _(par=par):
                pltpu.make_async_copy(
                    tableT.at[:, pl.ds(0, 128)], slab_v.at[par],
                    [sem_ga, sem_gb][par]).wait()

    def drain_rows(par):
        pltpu.make_async_copy(
            stage_hbm.at[pl.ds(0, 16)], rows_v.at[par],
            [sem_oa, sem_ob][par]).wait()

    fire(lo, jnp.int32(0))

    def tc_body(t, scc):
        bt = lax.rem(t - lo, jnp.int32(2))
        kd = jnp.right_shift(t - lo, 4)
        ksp = jnp.full((16,), 1, jnp.int32) * kd
        st_k = plsc.load_gather(starts_v, [ksp])[0]
        en_k = plsc.load_gather(starts_v, [ksp + 1])[0]

        @pl.when(t + 1 < hi)
        def _():
            fire(t + 1, lax.rem(t + 1 - lo, jnp.int32(2)))

        drain_slab(bt)

        def wbody(w, tp):
            woff = st_k + w * 16
            labv = lab2[pl.ds(woff, 16)]
            posv = pos2[pl.ds(woff, 16)]
            valid = (lane + woff) < en_k
            m = jnp.logical_and(jnp.right_shift(labv, 7) == t, valid)
            plsc.store_compressed(my_lab.at[pl.ds(tp, 16)], labv, mask=m)
            plsc.store_compressed(my_pos.at[pl.ds(tp, 16)], posv, mask=m)
            return tp + plsc.all_reduce_population_count(m)[0]

        nbw = lax.div(en_k - st_k + 15, jnp.int32(16))
        tcnt = lax.fori_loop(0, nbw, wbody, jnp.int32(0))

        def cbody(c2, scc2, bt=bt):
            b2 = lax.rem(scc2, jnp.int32(2))

            @pl.when(scc2 >= 2)
            def _():
                for par in range(2):
                    @pl.when(b2 == par)
                    def _(par=par):
                        drain_rows(par)

            base2 = c2 * 16
            rem16 = tcnt - base2
            gi = base2 + jnp.minimum(lane, rem16 - 1)
            pw = jnp.where(lane < rem16,
                           plsc.load_gather(my_pos, [gi]),
                           BATCH + lane)
            posw_v[b2, pl.ds(0, 16)] = pw
            bv = jnp.full((16,), 1, jnp.int32) * bt
            for s in range(16):
                @pl.when(s < rem16)
                def _(s=s, bv=bv, b2=b2):
                    labs = plsc.load_gather(
                        my_lab, [jnp.full((16,), base2 + s, jnp.int32)])[0]
                    phv = jnp.full((16,), 1, jnp.int32) * \
                        jnp.bitwise_and(labs, 127)
                    for kk in range(4):
                        cv = lane + kk * 16
                        vals = plsc.load_gather(slab_v, [bv, cv, phv])
                        rows_v[b2, s, pl.ds(kk * 16, 16)] = vals
            for par in range(2):
                @pl.when(b2 == par)
                def _(par=par):
                    pltpu.async_copy(
                        rows_v.at[par], stage_hbm.at[posw_v.at[par]],
                        [sem_oa, sem_ob][par])
            return scc2 + 1

        nch = lax.div(tcnt + 15, jnp.int32(16))
        scc = lax.fori_loop(0, nch, cbody, scc)
        return scc

    sccnt = lax.fori_loop(lo, hi, tc_body, jnp.int32(0))

    for par in range(2):
        @pl.when(jnp.logical_and(sccnt >= 2,
                                 lax.rem(sccnt, jnp.int32(2)) == par))
        def _(par=par):
            drain_rows(par)

        @pl.when(jnp.logical_and(sccnt >= 1,
                                 lax.rem(sccnt + 1, jnp.int32(2)) == par))
        def _(par=par):
            drain_rows(par)


@jax.jit
def _sc_gather(tableT, idx1d):
    mesh = plsc.VectorSubcoreMesh(core_axis_name="c", subcore_axis_name="s")
    fn = pl.kernel(
        _sc_body,
        out_type=jax.ShapeDtypeStruct((STAGE_ROWS, 128), jnp.float32),
        mesh=mesh,
        scratch_types=[
            pltpu.VMEM((BATCH,), jnp.int32),
            pltpu.VMEM((BATCH + 32,), jnp.int32),
            pltpu.VMEM((BATCH + 32,), jnp.int32),
            pltpu.VMEM((BATCH + 32,), jnp.int32),
            pltpu.VMEM((BATCH + 32,), jnp.int32),
            pltpu.VMEM((2, EMBED_DIM, 128), jnp.float32),
            pltpu.VMEM((2, 16, 128), jnp.float32),
            pltpu.VMEM((2, 16), jnp.int32),
            pltpu.VMEM((32,), jnp.int32),
            pltpu.SemaphoreType.DMA,
            pltpu.SemaphoreType.DMA,
            pltpu.SemaphoreType.DMA,
            pltpu.SemaphoreType.DMA,
        ],
        compiler_params=pltpu.CompilerParams(needs_layout_passes=False),
    )
    return fn(tableT, idx1d)


def _tc_linear_body(x_ref, w_ref, b_ref, o_ref):
    x = x_ref[...][:, :EMBED_DIM]
    s = x * jax.nn.sigmoid(x)
    o_ref[...] = (
        jax.lax.dot_general(w_ref[...], s, (((1,), (1,)), ((), ())),
                            preferred_element_type=jnp.float32) + b_ref[...]
    )


@jax.jit
def _tc_linear_t(stage, W, bcol):
    blk = 4096
    grid = (STAGE_ROWS // blk,)
    return pl.pallas_call(
        _tc_linear_body,
        grid=grid,
        in_specs=[
            pl.BlockSpec((blk, 128), lambda i: (i, 0)),
            pl.BlockSpec((EMBED_DIM, EMBED_DIM), lambda i: (0, 0)),
            pl.BlockSpec((EMBED_DIM, 1), lambda i: (0, 0)),
        ],
        out_specs=pl.BlockSpec((EMBED_DIM, blk), lambda i: (0, i)),
        out_shape=jax.ShapeDtypeStruct((EMBED_DIM, STAGE_ROWS), jnp.float32),
    )(stage, W, bcol)


def kernel(class_labels, table, W, b):
    idx1d = class_labels.astype(jnp.int32)
    stage = _sc_gather(table.T, idx1d)
    ot = _tc_linear_t(stage, W, b.reshape(EMBED_DIM, 1))
    return ot[:, :BATCH].T


# speedup vs baseline: 1.0141x; 1.0141x over previous
import functools

import jax
import jax.numpy as jnp
from jax import lax
from jax.experimental import pallas as pl
from jax.experimental.pallas import tpu as pltpu
from jax.experimental.pallas import tpu_sc as plsc

NUM_CLASSES = 1000000
EMBED_DIM = 64
BATCH = 16384

NC = 2
NS = 16
NW = NC * NS
NTB = (NUM_CLASSES + 127) // 128
TPW = (NTB + NW - 1) // NW
STAGE_ROWS = 20480


def _sc_body(tableT, idx_hbm, stage_hbm, all_idx, my_lab, my_pos, pos2,
             slab_v, rows_v, posw_v, starts_v, sem_g0, sem_g1, sem_g2,
             sem_g3, sem_g4, sem_g5, sem_oa, sem_ob):
    wid = lax.axis_index("s") * NC + lax.axis_index("c")
    lane = lax.iota(jnp.int32, 16)
    lo = wid * TPW
    hi = jnp.minimum(lo + TPW, NTB)
    pltpu.sync_copy(idx_hbm, all_idx.at[pl.ds(0, BATCH)])
    lab2 = all_idx
    sem_g = [sem_g0, sem_g1, sem_g2, sem_g3, sem_g4, sem_g5]

    def sc_body(q, ptr):
        for k in range(4):
            off = q * 64 + k * 16
            vec = all_idx[pl.ds(off, 16)]
            tb = jnp.right_shift(vec, 7)
            m = jnp.logical_and(tb >= lo, tb < hi)
            plsc.store_compressed(my_lab.at[pl.ds(ptr, 16)], vec, mask=m)
            plsc.store_compressed(my_pos.at[pl.ds(ptr, 16)], lane + off,
                                  mask=m)
            ptr = ptr + plsc.all_reduce_population_count(m)[0]
        return ptr

    mcnt = lax.fori_loop(0, BATCH // 64, sc_body, jnp.int32(0))
    nwin = lax.div(mcnt + 15, jnp.int32(16))

    starts = []
    gptr = jnp.int32(0)
    for k in range(16):
        starts.append(gptr)

        def b1(w, p, k=k):
            woff = w * 16
            labv = my_lab[pl.ds(woff, 16)]
            posv = my_pos[pl.ds(woff, 16)]
            valid = (lane + woff) < mcnt
            bk = jnp.right_shift(jnp.right_shift(labv, 7) - lo, 4)
            m = jnp.logical_and(bk == k, valid)
            plsc.store_compressed(lab2.at[pl.ds(p, 16)], labv, mask=m)
            plsc.store_compressed(pos2.at[pl.ds(p, 16)], posv, mask=m)
            return p + plsc.all_reduce_population_count(m)[0]

        gptr = lax.fori_loop(0, nwin, b1, gptr)
    starts.append(gptr)

    sv = jnp.zeros((16,), jnp.int32)
    for k in range(16):
        sv = jnp.where(lane == k, starts[k], sv)
    starts_v[pl.ds(0, 16)] = sv
    starts_v[pl.ds(16, 16)] = jnp.where(lane == 0, starts[16], 0)

    NSLAB = 6

    def fire(t, b):
        off = pl.multiple_of(t * 128, 128)
        for par in range(NSLAB):
            @pl.when(b == par)
            def _(par=par):
                pltpu.async_copy(
                    tableT.at[:, pl.ds(off, 128)], slab_v.at[par],
                    sem_g[par])

    def drain_slab(b):
        for par in range(NSLAB):
            @pl.when(b == par)
            def _(par=par):
                pltpu.make_async_copy(
                    tableT.at[:, pl.ds(0, 128)], slab_v.at[par],
                    sem_g[par]).wait()

    def drain_rows(par):
        pltpu.make_async_copy(
            stage_hbm.at[pl.ds(0, 16)], rows_v.at[par],
            [sem_oa, sem_ob][par]).wait()

    for j in range(5):
        @pl.when(lo + j < hi)
        def _(j=j):
            fire(lo + j, jnp.int32(j))

    def tc_body(t, scc):
        bt = lax.rem(t - lo, jnp.int32(6))
        kd = jnp.right_shift(t - lo, 4)
        ksp = jnp.full((16,), 1, jnp.int32) * kd
        st_k = plsc.load_gather(starts_v, [ksp])[0]
        en_k = plsc.load_gather(starts_v, [ksp + 1])[0]

        @pl.when(t + 5 < hi)
        def _():
            fire(t + 5, lax.rem(t + 5 - lo, jnp.int32(6)))

        drain_slab(bt)

        def wbody(w, tp):
            woff = st_k + w * 16
            labv = lab2[pl.ds(woff, 16)]
            posv = pos2[pl.ds(woff, 16)]
            valid = (lane + woff) < en_k
            m = jnp.logical_and(jnp.right_shift(labv, 7) == t, valid)
            plsc.store_compressed(my_lab.at[pl.ds(tp, 16)], labv, mask=m)
            plsc.store_compressed(my_pos.at[pl.ds(tp, 16)], posv, mask=m)
            return tp + plsc.all_reduce_population_count(m)[0]

        nbw = lax.div(en_k - st_k + 15, jnp.int32(16))
        tcnt = lax.fori_loop(0, nbw, wbody, jnp.int32(0))

        def cbody(c2, scc2, bt=bt):
            b2 = lax.rem(scc2, jnp.int32(2))

            @pl.when(scc2 >= 2)
            def _():
                for par in range(2):
                    @pl.when(b2 == par)
                    def _(par=par):
                        drain_rows(par)

            base2 = c2 * 16
            rem16 = tcnt - base2
            gi = base2 + jnp.minimum(lane, rem16 - 1)
            pw = jnp.where(lane < rem16,
                           plsc.load_gather(my_pos, [gi]),
                           BATCH + lane)
            posw_v[b2, pl.ds(0, 16)] = pw
            bv = jnp.full((16,), 1, jnp.int32) * bt
            for s in range(16):
                @pl.when(s < rem16)
                def _(s=s, bv=bv, b2=b2):
                    labs = plsc.load_gather(
                        my_lab, [jnp.full((16,), base2 + s, jnp.int32)])[0]
                    phv = jnp.full((16,), 1, jnp.int32) * \
                        jnp.bitwise_and(labs, 127)
                    for kk in range(4):
                        cv = lane + kk * 16
                        vals = plsc.load_gather(slab_v, [bv, cv, phv])
                        rows_v[b2, s, pl.ds(kk * 16, 16)] = vals
            for par in range(2):
                @pl.when(b2 == par)
                def _(par=par):
                    pltpu.async_copy(
                        rows_v.at[par], stage_hbm.at[posw_v.at[par]],
                        [sem_oa, sem_ob][par])
            return scc2 + 1

        nch = lax.div(tcnt + 15, jnp.int32(16))
        scc = lax.fori_loop(0, nch, cbody, scc)
        return scc

    sccnt = lax.fori_loop(lo, hi, tc_body, jnp.int32(0))

    for par in range(2):
        @pl.when(jnp.logical_and(sccnt >= 2,
                                 lax.rem(sccnt, jnp.int32(2)) == par))
        def _(par=par):
            drain_rows(par)

        @pl.when(jnp.logical_and(sccnt >= 1,
                                 lax.rem(sccnt + 1, jnp.int32(2)) == par))
        def _(par=par):
            drain_rows(par)


@jax.jit
def _sc_gather(tableT, idx1d):
    mesh = plsc.VectorSubcoreMesh(core_axis_name="c", subcore_axis_name="s")
    fn = pl.kernel(
        _sc_body,
        out_type=jax.ShapeDtypeStruct((STAGE_ROWS, 128), jnp.float32),
        mesh=mesh,
        scratch_types=[
            pltpu.VMEM((BATCH + 32,), jnp.int32),
            pltpu.VMEM((BATCH + 32,), jnp.int32),
            pltpu.VMEM((BATCH + 32,), jnp.int32),
            pltpu.VMEM((BATCH + 32,), jnp.int32),
            pltpu.VMEM((6, EMBED_DIM, 128), jnp.float32),
            pltpu.VMEM((2, 16, 128), jnp.float32),
            pltpu.VMEM((2, 16), jnp.int32),
            pltpu.VMEM((32,), jnp.int32),
            pltpu.SemaphoreType.DMA,
            pltpu.SemaphoreType.DMA,
            pltpu.SemaphoreType.DMA,
            pltpu.SemaphoreType.DMA,
            pltpu.SemaphoreType.DMA,
            pltpu.SemaphoreType.DMA,
            pltpu.SemaphoreType.DMA,
            pltpu.SemaphoreType.DMA,
        ],
        compiler_params=pltpu.CompilerParams(needs_layout_passes=False),
    )
    return fn(tableT, idx1d)


def _tc_linear_body(x_ref, w_ref, b_ref, o_ref):
    x = x_ref[...][:, :EMBED_DIM]
    s = x * jax.nn.sigmoid(x)
    o_ref[...] = (
        jax.lax.dot_general(w_ref[...], s, (((1,), (1,)), ((), ())),
                            preferred_element_type=jnp.float32) + b_ref[...]
    )


@jax.jit
def _tc_linear_t(stage, W, bcol):
    blk = 4096
    grid = (STAGE_ROWS // blk,)
    return pl.pallas_call(
        _tc_linear_body,
        grid=grid,
        in_specs=[
            pl.BlockSpec((blk, 128), lambda i: (i, 0)),
            pl.BlockSpec((EMBED_DIM, EMBED_DIM), lambda i: (0, 0)),
            pl.BlockSpec((EMBED_DIM, 1), lambda i: (0, 0)),
        ],
        out_specs=pl.BlockSpec((EMBED_DIM, blk), lambda i: (0, i)),
        out_shape=jax.ShapeDtypeStruct((EMBED_DIM, STAGE_ROWS), jnp.float32),
    )(stage, W, bcol)


def kernel(class_labels, table, W, b):
    idx1d = class_labels.astype(jnp.int32)
    stage = _sc_gather(table.T, idx1d)
    ot = _tc_linear_t(stage, W, b.reshape(EMBED_DIM, 1))
    return ot[:, :BATCH].T


# speedup vs baseline: 2.1761x; 2.1459x over previous
import functools

import jax
import jax.numpy as jnp
from jax import lax
from jax.experimental import pallas as pl
from jax.experimental.pallas import tpu as pltpu
from jax.experimental.pallas import tpu_sc as plsc

NUM_CLASSES = 1000000
EMBED_DIM = 64
BATCH = 16384

NC = 2
NS = 16
NW = NC * NS
B_PER_W = BATCH // NW
SB = 4
SB_LAB = B_PER_W // SB
NG = SB_LAB // 16


def _sc_gather_body(tableT, idx_hbm, outT_hbm, idx_v, slab_v, outT_v,
                    sem_g, sem_o):
    wid = lax.axis_index("s") * NC + lax.axis_index("c")
    base = wid * B_PER_W
    pltpu.sync_copy(idx_hbm.at[wid], idx_v)
    lane = lax.iota(jnp.int32, 16)

    l8 = jnp.bitwise_and(lane, 7)
    m8 = lane < 8

    def superblock(sb):
        for g in range(NG):
            gbase = sb * SB_LAB + g * 16
            vec = idx_v[pl.ds(gbase, 16)]
            tbv = jnp.right_shift(vec, 7)
            for sub in range(2):
                gh = []
                for l in range(8):
                    off = pl.multiple_of(tbv[sub * 8 + l] * 128, 128)
                    gh.append(pltpu.async_copy(
                        tableT.at[:, pl.ds(off, 128)],
                        slab_v.at[l], sem_g))
                for h in gh:
                    h.wait()
                pidx = jnp.full((16,), gbase + sub * 8, jnp.int32) + l8
                phs = jnp.bitwise_and(plsc.load_gather(idx_v, [pidx]), 127)
                for c in range(EMBED_DIM):
                    vals = plsc.load_gather(
                        slab_v, [l8, jnp.full((16,), c, jnp.int32), phs])
                    plsc.store_compressed(
                        outT_v.at[c, pl.ds(g * 16 + sub * 8, 16)], vals,
                        mask=m8)
        col0 = pl.multiple_of(base + sb * SB_LAB, 128)
        pltpu.sync_copy(outT_v.at[:, pl.ds(0, SB_LAB)],
                        outT_hbm.at[:, pl.ds(col0, SB_LAB)])

    pl.loop(0, SB)(superblock)


@jax.jit
def _sc_gather(tableT, idx2d):
    mesh = plsc.VectorSubcoreMesh(core_axis_name="c", subcore_axis_name="s")
    fn = pl.kernel(
        _sc_gather_body,
        out_type=jax.ShapeDtypeStruct((EMBED_DIM, BATCH), jnp.float32),
        mesh=mesh,
        scratch_types=[
            pltpu.VMEM((B_PER_W,), jnp.int32),
            pltpu.VMEM((8, EMBED_DIM, 128), jnp.float32),
            pltpu.VMEM((EMBED_DIM, SB_LAB + 16), jnp.float32),
            pltpu.SemaphoreType.DMA,
            pltpu.SemaphoreType.DMA,
        ],
        compiler_params=pltpu.CompilerParams(needs_layout_passes=False),
    )
    return fn(tableT, idx2d)


def _tc_linear_body(xt_ref, w_ref, b_ref, o_ref):
    x = xt_ref[...]
    s = x * jax.nn.sigmoid(x)
    o_ref[...] = (
        jax.lax.dot_general(w_ref[...], s, (((1,), (0,)), ((), ())),
                            preferred_element_type=jnp.float32) + b_ref[...]
    )


@jax.jit
def _tc_linear_t(xt, W, bcol):
    blk = 4096
    grid = (BATCH // blk,)
    return pl.pallas_call(
        _tc_linear_body,
        grid=grid,
        in_specs=[
            pl.BlockSpec((EMBED_DIM, blk), lambda i: (0, i)),
            pl.BlockSpec((EMBED_DIM, EMBED_DIM), lambda i: (0, 0)),
            pl.BlockSpec((EMBED_DIM, 1), lambda i: (0, 0)),
        ],
        out_specs=pl.BlockSpec((EMBED_DIM, blk), lambda i: (0, i)),
        out_shape=jax.ShapeDtypeStruct((EMBED_DIM, BATCH), jnp.float32),
    )(xt, W, bcol)


def kernel(class_labels, table, W, b):
    idx2d = class_labels.astype(jnp.int32).reshape(NW, B_PER_W)
    xt = _sc_gather(table.T, idx2d)
    ot = _tc_linear_t(xt, W, b.reshape(EMBED_DIM, 1))
    return ot.T


# speedup vs baseline: 2.3026x; 1.0581x over previous
import functools

import jax
import jax.numpy as jnp
from jax import lax
from jax.experimental import pallas as pl
from jax.experimental.pallas import tpu as pltpu
from jax.experimental.pallas import tpu_sc as plsc

NUM_CLASSES = 1000000
EMBED_DIM = 64
BATCH = 16384

NC = 2
NS = 16
NW = NC * NS
B_PER_W = BATCH // NW
SB = 4
SB_LAB = B_PER_W // SB
NG = SB_LAB // 16


def _sc_gather_body(tableT, idx_hbm, outT_hbm, idx_v, half_v, outT_v,
                    sem_g, sem_o):
    wid = lax.axis_index("s") * NC + lax.axis_index("c")
    base = wid * B_PER_W
    pltpu.sync_copy(idx_hbm.at[wid], idx_v)
    lane = lax.iota(jnp.int32, 16)

    def superblock(sb):
        for g in range(NG):
            vec = idx_v[pl.ds(sb * SB_LAB + g * 16, 16)]
            tbv = jnp.right_shift(vec, 7)
            phv = jnp.bitwise_and(vec, 127)
            for half in range(2):
                gh = []
                for l in range(16):
                    off = pl.multiple_of(tbv[l] * 128, 128)
                    gh.append(pltpu.async_copy(
                        tableT.at[pl.ds(half * 32, 32), pl.ds(off, 128)],
                        half_v.at[l], sem_g))
                for h in gh:
                    h.wait()
                for c in range(32):
                    vals = plsc.load_gather(
                        half_v, [lane, jnp.full((16,), c, jnp.int32), phv])
                    outT_v[half * 32 + c, pl.ds(g * 16, 16)] = vals
        col0 = pl.multiple_of(base + sb * SB_LAB, 128)
        pltpu.sync_copy(outT_v, outT_hbm.at[:, pl.ds(col0, SB_LAB)])

    pl.loop(0, SB)(superblock)


@jax.jit
def _sc_gather(tableT, idx2d):
    mesh = plsc.VectorSubcoreMesh(core_axis_name="c", subcore_axis_name="s")
    fn = pl.kernel(
        _sc_gather_body,
        out_type=jax.ShapeDtypeStruct((EMBED_DIM, BATCH), jnp.float32),
        mesh=mesh,
        scratch_types=[
            pltpu.VMEM((B_PER_W,), jnp.int32),
            pltpu.VMEM((16, 32, 128), jnp.float32),
            pltpu.VMEM((EMBED_DIM, SB_LAB), jnp.float32),
            pltpu.SemaphoreType.DMA,
            pltpu.SemaphoreType.DMA,
        ],
        compiler_params=pltpu.CompilerParams(needs_layout_passes=False),
    )
    return fn(tableT, idx2d)


def _tc_linear_body(xt_ref, w_ref, b_ref, o_ref):
    x = xt_ref[...]
    s = x * jax.nn.sigmoid(x)
    o_ref[...] = (
        jax.lax.dot_general(w_ref[...], s, (((1,), (0,)), ((), ())),
                            preferred_element_type=jnp.float32) + b_ref[...]
    )


@jax.jit
def _tc_linear_t(xt, W, bcol):
    blk = 4096
    grid = (BATCH // blk,)
    return pl.pallas_call(
        _tc_linear_body,
        grid=grid,
        in_specs=[
            pl.BlockSpec((EMBED_DIM, blk), lambda i: (0, i)),
            pl.BlockSpec((EMBED_DIM, EMBED_DIM), lambda i: (0, 0)),
            pl.BlockSpec((EMBED_DIM, 1), lambda i: (0, 0)),
        ],
        out_specs=pl.BlockSpec((EMBED_DIM, blk), lambda i: (0, i)),
        out_shape=jax.ShapeDtypeStruct((EMBED_DIM, BATCH), jnp.float32),
    )(xt, W, bcol)


def kernel(class_labels, table, W, b):
    idx2d = class_labels.astype(jnp.int32).reshape(NW, B_PER_W)
    xt = _sc_gather(table.T, idx2d)
    ot = _tc_linear_t(xt, W, b.reshape(EMBED_DIM, 1))
    return ot.T
